# baseline (speedup 1.0000x reference)
import jax
import jax.numpy as jnp
from jax import lax
from jax.experimental import pallas as pl
from jax.experimental.pallas import tpu as pltpu
from jax.experimental.pallas import tpu_sc as plsc

_B = 128
_TAB = 1000
_ROWS = 3 * 64 * 64
_G = 8
_BLK = _ROWS // _G


_NW = 32
_WROWS = _ROWS // _NW


def _sc_copy_body(src_hbm, dst_hbm, buf):
    w = lax.axis_index("s") * 2 + lax.axis_index("c")
    base = w * _WROWS
    pltpu.sync_copy(src_hbm.at[pl.ds(base, _WROWS)], buf)
    pltpu.sync_copy(buf, dst_hbm.at[pl.ds(base, _WROWS)])


_SC_COPY_CACHE = []


def _sc_copy():
    if not _SC_COPY_CACHE:
        _SC_COPY_CACHE.append(pl.kernel(
            _sc_copy_body,
            out_type=jax.ShapeDtypeStruct((_ROWS, _B), jnp.float32),
            mesh=plsc.VectorSubcoreMesh(core_axis_name="c",
                                        subcore_axis_name="s"),
            compiler_params=pltpu.CompilerParams(needs_layout_passes=False),
            scratch_types=[pltpu.VMEM((_WROWS, _B), jnp.float32)],
        ))
    return _SC_COPY_CACHE[0]


def _tc_body(t_sref, sac_sref, som_sref, x_ref, n_ref, o_ref,
             a_scr, s_scr):
    i = pl.program_id(0)

    @pl.when(i == 0)
    def _():
        lane = lax.broadcasted_iota(jnp.int32, (1, _B), 1)
        a_row = jnp.zeros((1, _B), jnp.float32)
        s_row = jnp.zeros((1, _B), jnp.float32)
        for j in range(_B):
            tj = t_sref[j]
            a_row = jnp.where(lane == j, sac_sref[tj], a_row)
            s_row = jnp.where(lane == j, som_sref[tj], s_row)
        a_scr[0:1, :] = a_row
        s_scr[0:1, :] = s_row

    a = a_scr[0:1, :]
    s = s_scr[0:1, :]
    o_ref[...] = a * x_ref[...] + s * n_ref[...]


def _tc_fma(t, sac, som, x2, n2):
    grid_spec = pltpu.PrefetchScalarGridSpec(
        num_scalar_prefetch=3,
        grid=(_G,),
        in_specs=[
            pl.BlockSpec((_BLK, _B), lambda i, *_: (i, 0)),
            pl.BlockSpec((_BLK, _B), lambda i, *_: (i, 0)),
        ],
        out_specs=pl.BlockSpec((_BLK, _B), lambda i, *_: (i, 0)),
        scratch_shapes=[
            pltpu.VMEM((8, _B), jnp.float32),
            pltpu.VMEM((8, _B), jnp.float32),
        ],
    )
    return pl.pallas_call(
        _tc_body,
        grid_spec=grid_spec,
        out_shape=jax.ShapeDtypeStruct((_ROWS, _B), jnp.float32),
    )(t, sac, som, x2, n2)


def kernel(x0, t, noise, sqrt_alphas_cumprod, sqrt_one_minus_alphas_cumprod):
    x2 = jnp.transpose(x0, (1, 2, 3, 0)).reshape(_ROWS, _B)
    n2 = jnp.transpose(noise, (1, 2, 3, 0)).reshape(_ROWS, _B)
    xt2 = _tc_fma(t.astype(jnp.int32), sqrt_alphas_cumprod,
                  sqrt_one_minus_alphas_cumprod, x2, n2)
    no2 = _sc_copy()(n2)
    x_t = jnp.transpose(xt2.reshape(3, 64, 64, _B), (3, 0, 1, 2))
    n_out = jnp.transpose(no2.reshape(3, 64, 64, _B), (3, 0, 1, 2))
    return (x_t, n_out)


# speedup vs baseline: 1.7558x; 1.7558x over previous
import jax
import jax.numpy as jnp
from jax import lax
from jax.experimental import pallas as pl
from jax.experimental.pallas import tpu as pltpu
from jax.experimental.pallas import tpu_sc as plsc

_B = 128
_TAB = 1000
_ROWS = 3 * 64 * 64
_G = 16
_BLK = _ROWS // _G


def _tc_body(t_sref, sac_sref, som_sref, x_ref, n_ref, o_ref, no_ref,
             a_scr, s_scr):
    i = pl.program_id(0)

    @pl.when(i == 0)
    def _():
        lane = lax.broadcasted_iota(jnp.int32, (1, _B), 1)
        a_row = jnp.zeros((1, _B), jnp.float32)
        s_row = jnp.zeros((1, _B), jnp.float32)
        for j in range(_B):
            tj = t_sref[j]
            a_row = jnp.where(lane == j, sac_sref[tj], a_row)
            s_row = jnp.where(lane == j, som_sref[tj], s_row)
        a_scr[0:1, :] = a_row
        s_scr[0:1, :] = s_row

    a = a_scr[0:1, :]
    s = s_scr[0:1, :]
    n = n_ref[...]
    o_ref[...] = a * x_ref[...] + s * n
    no_ref[...] = n


def _tc_fma(t, sac, som, x2, n2):
    grid_spec = pltpu.PrefetchScalarGridSpec(
        num_scalar_prefetch=3,
        grid=(_G,),
        in_specs=[
            pl.BlockSpec((_BLK, _B), lambda i, *_: (i, 0)),
            pl.BlockSpec((_BLK, _B), lambda i, *_: (i, 0)),
        ],
        out_specs=[
            pl.BlockSpec((_BLK, _B), lambda i, *_: (i, 0)),
            pl.BlockSpec((_BLK, _B), lambda i, *_: (i, 0)),
        ],
        scratch_shapes=[
            pltpu.VMEM((8, _B), jnp.float32),
            pltpu.VMEM((8, _B), jnp.float32),
        ],
    )
    return pl.pallas_call(
        _tc_body,
        grid_spec=grid_spec,
        out_shape=(jax.ShapeDtypeStruct((_ROWS, _B), jnp.float32),
                   jax.ShapeDtypeStruct((_ROWS, _B), jnp.float32)),
    )(t, sac, som, x2, n2)


def kernel(x0, t, noise, sqrt_alphas_cumprod, sqrt_one_minus_alphas_cumprod):
    x2 = jnp.transpose(x0, (1, 2, 3, 0)).reshape(_ROWS, _B)
    n2 = jnp.transpose(noise, (1, 2, 3, 0)).reshape(_ROWS, _B)
    xt2, no2 = _tc_fma(t.astype(jnp.int32), sqrt_alphas_cumprod,
                       sqrt_one_minus_alphas_cumprod, x2, n2)
    x_t = jnp.transpose(xt2.reshape(3, 64, 64, _B), (3, 0, 1, 2))
    n_out = jnp.transpose(no2.reshape(3, 64, 64, _B), (3, 0, 1, 2))
    return (x_t, n_out)


# speedup vs baseline: 2.5686x; 1.4629x over previous
import jax
import jax.numpy as jnp
from jax import lax
from jax.experimental import pallas as pl
from jax.experimental.pallas import tpu as pltpu
from jax.experimental.pallas import tpu_sc as plsc

_B = 128
_TAB = 1000
_ROWS = 3 * 64 * 64
_G = 4
_BLK = _ROWS // _G


def _tc_body(t_sref, sac_sref, som_sref, x_ref, n_ref, o_ref, no_ref,
             a_scr, s_scr):
    i = pl.program_id(0)

    @pl.when(i == 0)
    def _():
        lane = lax.broadcasted_iota(jnp.int32, (1, _B), 1)
        a_row = jnp.zeros((1, _B), jnp.float32)
        s_row = jnp.zeros((1, _B), jnp.float32)
        for j in range(_B):
            tj = t_sref[j]
            a_row = jnp.where(lane == j, sac_sref[tj], a_row)
            s_row = jnp.where(lane == j, som_sref[tj], s_row)
        a_scr[0:1, :] = a_row
        s_scr[0:1, :] = s_row

    a = a_scr[0:1, :]
    s = s_scr[0:1, :]
    n = n_ref[...]
    o_ref[...] = a * x_ref[...] + s * n
    no_ref[...] = n


def _tc_fma(t, sac, som, x2, n2):
    grid_spec = pltpu.PrefetchScalarGridSpec(
        num_scalar_prefetch=3,
        grid=(_G,),
        in_specs=[
            pl.BlockSpec((_BLK, _B), lambda i, *_: (i, 0)),
            pl.BlockSpec((_BLK, _B), lambda i, *_: (i, 0)),
        ],
        out_specs=[
            pl.BlockSpec((_BLK, _B), lambda i, *_: (i, 0)),
            pl.BlockSpec((_BLK, _B), lambda i, *_: (i, 0)),
        ],
        scratch_shapes=[
            pltpu.VMEM((8, _B), jnp.float32),
            pltpu.VMEM((8, _B), jnp.float32),
        ],
    )
    return pl.pallas_call(
        _tc_body,
        grid_spec=grid_spec,
        out_shape=(jax.ShapeDtypeStruct((_ROWS, _B), jnp.float32),
                   jax.ShapeDtypeStruct((_ROWS, _B), jnp.float32)),
    )(t, sac, som, x2, n2)


def kernel(x0, t, noise, sqrt_alphas_cumprod, sqrt_one_minus_alphas_cumprod):
    x2 = jnp.transpose(x0, (1, 2, 3, 0)).reshape(_ROWS, _B)
    n2 = jnp.transpose(noise, (1, 2, 3, 0)).reshape(_ROWS, _B)
    xt2, no2 = _tc_fma(t.astype(jnp.int32), sqrt_alphas_cumprod,
                       sqrt_one_minus_alphas_cumprod, x2, n2)
    x_t = jnp.transpose(xt2.reshape(3, 64, 64, _B), (3, 0, 1, 2))
    n_out = jnp.transpose(no2.reshape(3, 64, 64, _B), (3, 0, 1, 2))
    return (x_t, n_out)


# speedup vs baseline: 3.0146x; 1.1736x over previous
import jax
import jax.numpy as jnp
from jax import lax
from jax.experimental import pallas as pl
from jax.experimental.pallas import tpu as pltpu
from jax.experimental.pallas import tpu_sc as plsc

_B = 128
_TAB = 1000
_ROWS = 3 * 64 * 64
_G = 2
_BLK = _ROWS // _G


def _tc_body(t_sref, sac_sref, som_sref, x_ref, n_ref, o_ref, no_ref,
             a_scr, s_scr):
    i = pl.program_id(0)

    @pl.when(i == 0)
    def _():
        lane = lax.broadcasted_iota(jnp.int32, (1, _B), 1)
        a_row = jnp.zeros((1, _B), jnp.float32)
        s_row = jnp.zeros((1, _B), jnp.float32)
        for j in range(_B):
            tj = t_sref[j]
            a_row = jnp.where(lane == j, sac_sref[tj], a_row)
            s_row = jnp.where(lane == j, som_sref[tj], s_row)
        a_scr[0:1, :] = a_row
        s_scr[0:1, :] = s_row

    a = a_scr[0:1, :]
    s = s_scr[0:1, :]
    n = n_ref[...]
    o_ref[...] = a * x_ref[...] + s * n
    no_ref[...] = n


def _tc_fma(t, sac, som, x2, n2):
    grid_spec = pltpu.PrefetchScalarGridSpec(
        num_scalar_prefetch=3,
        grid=(_G,),
        in_specs=[
            pl.BlockSpec((_BLK, _B), lambda i, *_: (i, 0)),
            pl.BlockSpec((_BLK, _B), lambda i, *_: (i, 0)),
        ],
        out_specs=[
            pl.BlockSpec((_BLK, _B), lambda i, *_: (i, 0)),
            pl.BlockSpec((_BLK, _B), lambda i, *_: (i, 0)),
        ],
        scratch_shapes=[
            pltpu.VMEM((8, _B), jnp.float32),
            pltpu.VMEM((8, _B), jnp.float32),
        ],
    )
    return pl.pallas_call(
        _tc_body,
        grid_spec=grid_spec,
        out_shape=(jax.ShapeDtypeStruct((_ROWS, _B), jnp.float32),
                   jax.ShapeDtypeStruct((_ROWS, _B), jnp.float32)),
    )(t, sac, som, x2, n2)


def kernel(x0, t, noise, sqrt_alphas_cumprod, sqrt_one_minus_alphas_cumprod):
    x2 = jnp.transpose(x0, (1, 2, 3, 0)).reshape(_ROWS, _B)
    n2 = jnp.transpose(noise, (1, 2, 3, 0)).reshape(_ROWS, _B)
    xt2, no2 = _tc_fma(t.astype(jnp.int32), sqrt_alphas_cumprod,
                       sqrt_one_minus_alphas_cumprod, x2, n2)
    x_t = jnp.transpose(xt2.reshape(3, 64, 64, _B), (3, 0, 1, 2))
    n_out = jnp.transpose(no2.reshape(3, 64, 64, _B), (3, 0, 1, 2))
    return (x_t, n_out)
